# baseline (speedup 1.0000x reference)
import functools

import jax
import jax.numpy as jnp
from jax import lax
from jax.experimental import pallas as pl
from jax.experimental.pallas import tpu as pltpu
from jax.experimental.pallas import tpu_sc as plsc

N = 10000
DEG = 32
DIN = 128
DOUT = 128
HALF = DOUT // 2

NC = 2
NS = 16
L = 16
NW = NC * NS
NPW = 320
NPAD = NW * NPW
NB = 4
ROWS = NB * DEG
NBATCH = NPW // NB
K = 8


def _mm_body(f_ref, w_ref, b_ref, o_ref):
    y = jnp.dot(f_ref[...].astype(jnp.bfloat16), w_ref[...].astype(jnp.bfloat16),
                preferred_element_type=jnp.float32)
    y = jnp.maximum(y + b_ref[...], 0.0) * (1.0 / DEG)
    a_bits = lax.bitcast_convert_type(y[:, :HALF], jnp.int32)
    b_bits = lax.bitcast_convert_type(y[:, HALF:], jnp.int32)
    a_r = a_bits + 0x7FFF + jnp.bitwise_and(jnp.right_shift(a_bits, 16), 1)
    b_r = b_bits + 0x7FFF + jnp.bitwise_and(jnp.right_shift(b_bits, 16), 1)
    a_top = jnp.bitwise_and(a_r, jnp.int32(-65536))
    b_top = jnp.bitwise_and(jnp.right_shift(b_r, 16), jnp.int32(0xFFFF))
    o_ref[...] = jnp.bitwise_or(a_top, b_top)


def _linear_relu_pack(feats, W, b2):
    blk = 2560
    return pl.pallas_call(
        _mm_body,
        grid=(NPAD // blk,),
        in_specs=[
            pl.BlockSpec((blk, DIN), lambda i: (i, 0)),
            pl.BlockSpec((DIN, DOUT), lambda i: (0, 0)),
            pl.BlockSpec((1, DOUT), lambda i: (0, 0)),
        ],
        out_specs=pl.BlockSpec((blk, HALF), lambda i: (i, 0)),
        out_shape=jax.ShapeDtypeStruct((NPAD, HALF), jnp.int32),
    )(feats, W, b2)


def _sc_body(y_hbm, eidx_hbm, out_hbm, idx_v, y_sh, *rest):
    bufs = rest[:K]
    stages = rest[K:2 * K]
    sems = rest[2 * K:]
    c = lax.axis_index("c")
    s = lax.axis_index("s")
    wid = s * NC + c
    base = jnp.minimum(wid * NPW, N - NPW)

    rpt = NPAD // NS
    pltpu.sync_copy(y_hbm.at[pl.ds(s * rpt, rpt)], y_sh.at[pl.ds(s * rpt, rpt)])

    pltpu.sync_copy(eidx_hbm.at[pl.ds(base * DEG, NPW * DEG)], idx_v)
    plsc.subcore_barrier()

    def gather_start(bi, buf, sem):
        pltpu.async_copy(y_sh.at[idx_v.at[pl.ds(bi * ROWS, ROWS)]], buf, sem)

    def gather_wait(bi, buf, sem):
        pltpu.make_async_copy(
            y_sh.at[idx_v.at[pl.ds(bi * ROWS, ROWS)]], buf, sem
        ).wait()

    SH16 = jnp.full((L,), 16, dtype=jnp.int32)

    def load2(rows_v, r, col):
        w = rows_v[r, col]
        a = lax.bitcast_convert_type(w, jnp.float32)
        b = lax.bitcast_convert_type(jnp.left_shift(w, SH16), jnp.float32)
        return a, b

    def process(bi, rows_v, stage_v):
        for ni in range(NB):
            r0 = ni * DEG
            for ch in range(HALF // L):
                col = pl.ds(ch * L, L)
                a0, b0 = load2(rows_v, r0 + 0, col)
                a1, b1 = load2(rows_v, r0 + 1, col)
                for j in range(2, DEG, 2):
                    x0, y0 = load2(rows_v, r0 + j + 0, col)
                    x1, y1 = load2(rows_v, r0 + j + 1, col)
                    a0 = a0 + x0
                    b0 = b0 + y0
                    a1 = a1 + x1
                    b1 = b1 + y1
                stage_v[ni, pl.ds(ch * L, L)] = a0 + a1
                stage_v[ni, pl.ds(HALF + ch * L, L)] = b0 + b1
        pltpu.sync_copy(stage_v, out_hbm.at[pl.ds(base + bi * NB, NB)])

    for j in range(K):
        gather_start(j, bufs[j], sems[j])

    def group_body(g, carry):
        b0 = g * K
        for j in range(K):
            bi = b0 + j
            gather_wait(bi, bufs[j], sems[j])
            process(bi, bufs[j], stages[j])

            @pl.when(bi + K < NBATCH)
            def _():
                gather_start(bi + K, bufs[j], sems[j])
        return carry

    lax.fori_loop(0, NBATCH // K, group_body, 0)


def _sc_gather(y_packed, eidx):
    mesh = plsc.VectorSubcoreMesh(core_axis_name="c", subcore_axis_name="s")
    fn = functools.partial(
        pl.kernel,
        mesh=mesh,
        out_type=jax.ShapeDtypeStruct((N, DOUT), jnp.float32),
        compiler_params=pltpu.CompilerParams(use_tc_tiling_on_sc=False),
        scratch_types=(
            [pltpu.VMEM((NPW * DEG,), jnp.int32)]
            + [pltpu.VMEM_SHARED((NPAD, HALF), jnp.int32)]
            + [pltpu.VMEM((ROWS, HALF), jnp.int32) for _ in range(K)]
            + [pltpu.VMEM((NB, DOUT), jnp.float32) for _ in range(K)]
            + [pltpu.SemaphoreType.DMA for _ in range(K)]
        ),
    )(_sc_body)
    return fn(y_packed, eidx)


@jax.jit
def _impl(feats, edge_dict, W, b):
    y_packed = _linear_relu_pack(feats, W, b.reshape(1, DOUT))
    return _sc_gather(y_packed, edge_dict.reshape(-1))


def kernel(ids, feats, edge_dict, G, ite, W, b):
    return _impl(feats, edge_dict, W, b)


# speedup vs baseline: 1.1046x; 1.1046x over previous
import functools

import jax
import jax.numpy as jnp
from jax import lax
from jax.experimental import pallas as pl
from jax.experimental.pallas import tpu as pltpu
from jax.experimental.pallas import tpu_sc as plsc

N = 10000
DEG = 32
DIN = 128
DOUT = 128
HALF = DOUT // 2

NC = 2
NS = 16
L = 16
NW = NC * NS
NPW = 320
NPAD = NW * NPW
NB = 4
ROWS = NB * DEG
NBATCH = NPW // NB
K = 4


def _mm_body(f_ref, w_ref, b_ref, o_ref):
    y = jnp.dot(f_ref[...].astype(jnp.bfloat16), w_ref[...].astype(jnp.bfloat16),
                preferred_element_type=jnp.float32)
    y = jnp.maximum(y + b_ref[...], 0.0) * (1.0 / DEG)
    a_bits = lax.bitcast_convert_type(y[:, :HALF], jnp.int32)
    b_bits = lax.bitcast_convert_type(y[:, HALF:], jnp.int32)
    a_r = a_bits + 0x7FFF + jnp.bitwise_and(jnp.right_shift(a_bits, 16), 1)
    b_r = b_bits + 0x7FFF + jnp.bitwise_and(jnp.right_shift(b_bits, 16), 1)
    a_top = jnp.bitwise_and(a_r, jnp.int32(-65536))
    b_top = jnp.bitwise_and(jnp.right_shift(b_r, 16), jnp.int32(0xFFFF))
    o_ref[...] = jnp.bitwise_or(a_top, b_top)


def _linear_relu_pack(feats, W, b2):
    blk = 2560
    return pl.pallas_call(
        _mm_body,
        grid=(NPAD // blk,),
        in_specs=[
            pl.BlockSpec((blk, DIN), lambda i: (i, 0)),
            pl.BlockSpec((DIN, DOUT), lambda i: (0, 0)),
            pl.BlockSpec((1, DOUT), lambda i: (0, 0)),
        ],
        out_specs=pl.BlockSpec((blk, HALF), lambda i: (i, 0)),
        out_shape=jax.ShapeDtypeStruct((NPAD, HALF), jnp.int32),
    )(feats, W, b2)


def _sc_body(y_hbm, eidx_hbm, out_hbm, idx_v, y_sh, *rest):
    bufs = rest[:K]
    stages = rest[K:2 * K]
    sems = rest[2 * K:]
    c = lax.axis_index("c")
    s = lax.axis_index("s")
    wid = s * NC + c
    base = jnp.minimum(wid * NPW, N - NPW)

    rpt = NPAD // NS
    pltpu.sync_copy(y_hbm.at[pl.ds(s * rpt, rpt)], y_sh.at[pl.ds(s * rpt, rpt)])

    pltpu.sync_copy(eidx_hbm.at[pl.ds(base * DEG, NPW * DEG)], idx_v)
    plsc.subcore_barrier()

    def gather_start(bi, buf, sem):
        pltpu.async_copy(y_sh.at[idx_v.at[pl.ds(bi * ROWS, ROWS)]], buf, sem)

    def gather_wait(bi, buf, sem):
        pltpu.make_async_copy(
            y_sh.at[idx_v.at[pl.ds(bi * ROWS, ROWS)]], buf, sem
        ).wait()

    SH16 = jnp.full((L,), 16, dtype=jnp.int32)

    def load2(rows_v, r, col):
        w = rows_v[r, col]
        a = lax.bitcast_convert_type(w, jnp.float32)
        b = lax.bitcast_convert_type(jnp.left_shift(w, SH16), jnp.float32)
        return a, b

    def process(bi, rows_v, stage_v):
        for ni in range(NB):
            r0 = ni * DEG
            for ch in range(HALF // L):
                col = pl.ds(ch * L, L)
                a0, b0 = load2(rows_v, r0 + 0, col)
                a1, b1 = load2(rows_v, r0 + 1, col)
                for j in range(2, DEG, 2):
                    x0, y0 = load2(rows_v, r0 + j + 0, col)
                    x1, y1 = load2(rows_v, r0 + j + 1, col)
                    a0 = a0 + x0
                    b0 = b0 + y0
                    a1 = a1 + x1
                    b1 = b1 + y1
                stage_v[ni, pl.ds(ch * L, L)] = a0 + a1
                stage_v[ni, pl.ds(HALF + ch * L, L)] = b0 + b1
        pltpu.sync_copy(stage_v, out_hbm.at[pl.ds(base + bi * NB, NB)])

    for j in range(K):
        gather_start(j, bufs[j], sems[j])

    def group_body(g, carry):
        b0 = g * K
        for j in range(K):
            bi = b0 + j
            gather_wait(bi, bufs[j], sems[j])
            process(bi, bufs[j], stages[j])

            @pl.when(bi + K < NBATCH)
            def _():
                gather_start(bi + K, bufs[j], sems[j])
        return carry

    lax.fori_loop(0, NBATCH // K, group_body, 0)


def _sc_gather(y_packed, eidx):
    mesh = plsc.VectorSubcoreMesh(core_axis_name="c", subcore_axis_name="s")
    fn = functools.partial(
        pl.kernel,
        mesh=mesh,
        out_type=jax.ShapeDtypeStruct((N, DOUT), jnp.float32),
        compiler_params=pltpu.CompilerParams(use_tc_tiling_on_sc=False),
        scratch_types=(
            [pltpu.VMEM((NPW * DEG,), jnp.int32)]
            + [pltpu.VMEM_SHARED((NPAD, HALF), jnp.int32)]
            + [pltpu.VMEM((ROWS, HALF), jnp.int32) for _ in range(K)]
            + [pltpu.VMEM((NB, DOUT), jnp.float32) for _ in range(K)]
            + [pltpu.SemaphoreType.DMA for _ in range(K)]
        ),
    )(_sc_body)
    return fn(y_packed, eidx)


@jax.jit
def _impl(feats, edge_dict, W, b):
    y_packed = _linear_relu_pack(feats, W, b.reshape(1, DOUT))
    return _sc_gather(y_packed, edge_dict.reshape(-1))


def kernel(ids, feats, edge_dict, G, ite, W, b):
    return _impl(feats, edge_dict, W, b)
